# Initial kernel scaffold; baseline (speedup 1.0000x reference)
import jax
import jax.numpy as jnp
from jax.experimental import pallas as pl


def kernel(hidden_states, visual_token_mask, text_gate_w, vision_gate_w, e_score_correction_bias, text_wg, text_wu, text_wd, vision_wg, vision_wu, vision_wd, shared_wg, shared_wu, shared_wd):
    raise NotImplementedError("write your pallas kernel here")


# speedup vs baseline: 1.6828x; 1.6828x over previous
import functools

import jax
import jax.numpy as jnp
from jax import lax
from jax.experimental import pallas as pl
from jax.experimental.pallas import tpu as pltpu

T = 2048
D = 1024
DFF = 512
E = 8
TB = 256
NT = T // TB


def _gating_block(xf, gates, bias_t, bias_v, mask_blk):
    logits = lax.dot_general(
        xf, gates, (((1,), (0,)), ((), ())),
        preferred_element_type=jnp.float32,
        precision=lax.Precision.DEFAULT,
    )

    def bank(l, brow):
        m = jnp.max(l, axis=1, keepdims=True)
        ex = jnp.exp(l - m)
        s = ex / jnp.sum(ex, axis=1, keepdims=True)
        b = s + brow
        iota = lax.broadcasted_iota(jnp.int32, (TB, E), 1)
        m1 = jnp.max(b, axis=1, keepdims=True)
        idx1 = jnp.min(jnp.where(b == m1, iota, E), axis=1, keepdims=True)
        bm = jnp.where(iota == idx1, -jnp.inf, b)
        m2 = jnp.max(bm, axis=1, keepdims=True)
        idx2 = jnp.min(jnp.where(bm == m2, iota, E), axis=1, keepdims=True)
        w1 = jnp.sum(jnp.where(iota == idx1, s, 0.0), axis=1, keepdims=True)
        w2 = jnp.sum(jnp.where(iota == idx2, s, 0.0), axis=1, keepdims=True)
        ws = w1 + w2
        return (iota == idx1) * (w1 / ws) + (iota == idx2) * (w2 / ws)

    wt = bank(logits[:, :E], bias_t) * (1.0 - mask_blk)
    wv = bank(logits[:, E:], bias_v) * mask_blk
    return jnp.concatenate([wt, wv], axis=1)


def _moe_body(x_ref, maskf_ref, gates_ref, bias_ref,
              twg_ref, twu_ref, twd_ref,
              vwg_ref, vwu_ref, vwd_ref,
              swg_ref, swu_ref, swd_ref,
              out_ref, xb_scr, w_scr, acc_ref):
    e = pl.program_id(0)
    t = pl.program_id(1)
    sl = pl.ds(t * TB, TB)

    @pl.when(e == 0)
    def _():
        xf = x_ref[...]
        xb_scr[sl, :] = xf.astype(jnp.bfloat16)
        mask_blk = maskf_ref[sl, :]
        w_scr[sl, :] = _gating_block(
            xf, gates_ref[...], bias_ref[0:1, :], bias_ref[1:2, :], mask_blk)
        acc_ref[sl, :] = jnp.zeros((TB, D), jnp.float32)

    xb = xb_scr[sl, :]
    iota16 = lax.broadcasted_iota(jnp.int32, (TB, 2 * E), 1)
    wcol = jnp.sum(w_scr[sl, :] * (iota16 == e).astype(jnp.float32),
                   axis=1, keepdims=True)
    wcol = jnp.where(e == 2 * E, 1.0, wcol)

    def mlp_acc(wg, wu, wd):
        g = lax.dot_general(xb, wg, (((1,), (0,)), ((), ())),
                            preferred_element_type=jnp.float32)
        u = lax.dot_general(xb, wu, (((1,), (0,)), ((), ())),
                            preferred_element_type=jnp.float32)
        a = (g * jax.nn.sigmoid(g)) * u
        h = lax.dot_general(a.astype(jnp.bfloat16), wd,
                            (((1,), (0,)), ((), ())),
                            preferred_element_type=jnp.float32)
        acc_ref[sl, :] += wcol * h

    @pl.when(e < E)
    def _():
        mlp_acc(twg_ref[0].astype(jnp.bfloat16),
                twu_ref[0].astype(jnp.bfloat16),
                twd_ref[0].astype(jnp.bfloat16))

    @pl.when((e >= E) & (e < 2 * E))
    def _():
        mlp_acc(vwg_ref[0].astype(jnp.bfloat16),
                vwu_ref[0].astype(jnp.bfloat16),
                vwd_ref[0].astype(jnp.bfloat16))

    @pl.when(e == 2 * E)
    def _():
        mlp_acc(swg_ref[...].astype(jnp.bfloat16),
                swu_ref[...].astype(jnp.bfloat16),
                swd_ref[...].astype(jnp.bfloat16))
        out_ref[...] = acc_ref[sl, :]


def kernel(hidden_states, visual_token_mask, text_gate_w, vision_gate_w,
           e_score_correction_bias, text_wg, text_wu, text_wd,
           vision_wg, vision_wu, vision_wd, shared_wg, shared_wu, shared_wd):
    orig_shape = hidden_states.shape
    x = hidden_states.reshape(T, D)
    maskf = visual_token_mask.reshape(T, 1).astype(jnp.float32)
    gates = jnp.concatenate([text_gate_w, vision_gate_w], axis=1)

    grid = (2 * E + 1, NT)

    out = pl.pallas_call(
        _moe_body,
        grid=grid,
        in_specs=[
            pl.BlockSpec((TB, D), lambda e, t: (jnp.where(e == 0, t, 0), 0)),
            pl.BlockSpec((T, 1), lambda e, t: (0, 0)),
            pl.BlockSpec((D, 2 * E), lambda e, t: (0, 0)),
            pl.BlockSpec((2, E), lambda e, t: (0, 0)),
            pl.BlockSpec((1, D, DFF), lambda e, t: (jnp.minimum(e, E - 1), 0, 0)),
            pl.BlockSpec((1, D, DFF), lambda e, t: (jnp.minimum(e, E - 1), 0, 0)),
            pl.BlockSpec((1, DFF, D), lambda e, t: (jnp.minimum(e, E - 1), 0, 0)),
            pl.BlockSpec((1, D, DFF),
                         lambda e, t: (jnp.clip(e - E, 0, E - 1), 0, 0)),
            pl.BlockSpec((1, D, DFF),
                         lambda e, t: (jnp.clip(e - E, 0, E - 1), 0, 0)),
            pl.BlockSpec((1, DFF, D),
                         lambda e, t: (jnp.clip(e - E, 0, E - 1), 0, 0)),
            pl.BlockSpec((D, DFF), lambda e, t: (0, 0)),
            pl.BlockSpec((D, DFF), lambda e, t: (0, 0)),
            pl.BlockSpec((DFF, D), lambda e, t: (0, 0)),
        ],
        out_specs=pl.BlockSpec(
            (TB, D), lambda e, t: (jnp.where(e == 2 * E, t, 0), 0)),
        out_shape=jax.ShapeDtypeStruct((T, D), jnp.float32),
        scratch_shapes=[
            pltpu.VMEM((T, D), jnp.bfloat16),
            pltpu.VMEM((T, 2 * E), jnp.float32),
            pltpu.VMEM((T, D), jnp.float32),
        ],
        compiler_params=pltpu.CompilerParams(
            dimension_semantics=("arbitrary", "arbitrary")),
    )(x, maskf, gates, e_score_correction_bias,
      text_wg, text_wu, text_wd, vision_wg, vision_wu, vision_wd,
      shared_wg, shared_wu, shared_wd)

    return out.reshape(orig_shape)


# speedup vs baseline: 1.7259x; 1.0256x over previous
import functools

import jax
import jax.numpy as jnp
from jax import lax
from jax.experimental import pallas as pl
from jax.experimental.pallas import tpu as pltpu

T = 2048
D = 1024
DFF = 512
E = 8
TB = 256
NT = T // TB


def _gating_block(xf, gates, bias_t, bias_v, mask_blk):
    logits = lax.dot_general(
        xf, gates, (((1,), (0,)), ((), ())),
        preferred_element_type=jnp.float32,
        precision=lax.Precision.DEFAULT,
    )

    def bank(l, brow):
        m = jnp.max(l, axis=1, keepdims=True)
        ex = jnp.exp(l - m)
        s = ex / jnp.sum(ex, axis=1, keepdims=True)
        b = s + brow
        iota = lax.broadcasted_iota(jnp.int32, (TB, E), 1)
        m1 = jnp.max(b, axis=1, keepdims=True)
        idx1 = jnp.min(jnp.where(b == m1, iota, E), axis=1, keepdims=True)
        bm = jnp.where(iota == idx1, -jnp.inf, b)
        m2 = jnp.max(bm, axis=1, keepdims=True)
        idx2 = jnp.min(jnp.where(bm == m2, iota, E), axis=1, keepdims=True)
        w1 = jnp.sum(jnp.where(iota == idx1, s, 0.0), axis=1, keepdims=True)
        w2 = jnp.sum(jnp.where(iota == idx2, s, 0.0), axis=1, keepdims=True)
        ws = w1 + w2
        return (iota == idx1) * (w1 / ws) + (iota == idx2) * (w2 / ws)

    wt = bank(logits[:, :E], bias_t) * (1.0 - mask_blk)
    wv = bank(logits[:, E:], bias_v) * mask_blk
    return jnp.concatenate([wt, wv], axis=1)


def _moe_body(x_ref, maskf_ref, gates_ref, bias_ref,
              twg_ref, twu_ref, twd_ref,
              vwg_ref, vwu_ref, vwd_ref,
              swg_ref, swu_ref, swd_ref,
              out_ref, xb_scr, w_scr, acc_ref,
              wgb_scr, wub_scr, wdb_scr):
    e = pl.program_id(0)
    t = pl.program_id(1)
    sl = pl.ds(t * TB, TB)

    @pl.when(e == 0)
    def _():
        xf = x_ref[...]
        xb_scr[sl, :] = xf.astype(jnp.bfloat16)
        mask_blk = maskf_ref[sl, :]
        w_scr[sl, :] = _gating_block(
            xf, gates_ref[...], bias_ref[0:1, :], bias_ref[1:2, :], mask_blk)
        acc_ref[sl, :] = jnp.zeros((TB, D), jnp.float32)

    @pl.when(t == 0)
    def _():
        @pl.when(e < E)
        def _():
            wgb_scr[...] = twg_ref[0].astype(jnp.bfloat16)
            wub_scr[...] = twu_ref[0].astype(jnp.bfloat16)
            wdb_scr[...] = twd_ref[0].astype(jnp.bfloat16)

        @pl.when((e >= E) & (e < 2 * E))
        def _():
            wgb_scr[...] = vwg_ref[0].astype(jnp.bfloat16)
            wub_scr[...] = vwu_ref[0].astype(jnp.bfloat16)
            wdb_scr[...] = vwd_ref[0].astype(jnp.bfloat16)

        @pl.when(e == 2 * E)
        def _():
            wgb_scr[...] = swg_ref[...].astype(jnp.bfloat16)
            wub_scr[...] = swu_ref[...].astype(jnp.bfloat16)
            wdb_scr[...] = swd_ref[...].astype(jnp.bfloat16)

    xb = xb_scr[sl, :]
    iota16 = lax.broadcasted_iota(jnp.int32, (TB, 2 * E), 1)
    wcol = jnp.sum(w_scr[sl, :] * (iota16 == e).astype(jnp.float32),
                   axis=1, keepdims=True)
    wcol = jnp.where(e == 2 * E, 1.0, wcol)

    g = lax.dot_general(xb, wgb_scr[...], (((1,), (0,)), ((), ())),
                        preferred_element_type=jnp.float32)
    u = lax.dot_general(xb, wub_scr[...], (((1,), (0,)), ((), ())),
                        preferred_element_type=jnp.float32)
    a = (g * jax.nn.sigmoid(g)) * u
    h = lax.dot_general(a.astype(jnp.bfloat16), wdb_scr[...],
                        (((1,), (0,)), ((), ())),
                        preferred_element_type=jnp.float32)
    acc_ref[sl, :] += wcol * h

    @pl.when(e == 2 * E)
    def _():
        out_ref[...] = acc_ref[sl, :]


def kernel(hidden_states, visual_token_mask, text_gate_w, vision_gate_w,
           e_score_correction_bias, text_wg, text_wu, text_wd,
           vision_wg, vision_wu, vision_wd, shared_wg, shared_wu, shared_wd):
    orig_shape = hidden_states.shape
    x = hidden_states.reshape(T, D)
    maskf = visual_token_mask.reshape(T, 1).astype(jnp.float32)
    gates = jnp.concatenate([text_gate_w, vision_gate_w], axis=1)

    grid = (2 * E + 1, NT)

    out = pl.pallas_call(
        _moe_body,
        grid=grid,
        in_specs=[
            pl.BlockSpec((TB, D), lambda e, t: (jnp.where(e == 0, t, 0), 0)),
            pl.BlockSpec((T, 1), lambda e, t: (0, 0)),
            pl.BlockSpec((D, 2 * E), lambda e, t: (0, 0)),
            pl.BlockSpec((2, E), lambda e, t: (0, 0)),
            pl.BlockSpec((1, D, DFF), lambda e, t: (jnp.minimum(e, E - 1), 0, 0)),
            pl.BlockSpec((1, D, DFF), lambda e, t: (jnp.minimum(e, E - 1), 0, 0)),
            pl.BlockSpec((1, DFF, D), lambda e, t: (jnp.minimum(e, E - 1), 0, 0)),
            pl.BlockSpec((1, D, DFF),
                         lambda e, t: (jnp.clip(e - E, 0, E - 1), 0, 0)),
            pl.BlockSpec((1, D, DFF),
                         lambda e, t: (jnp.clip(e - E, 0, E - 1), 0, 0)),
            pl.BlockSpec((1, DFF, D),
                         lambda e, t: (jnp.clip(e - E, 0, E - 1), 0, 0)),
            pl.BlockSpec((D, DFF), lambda e, t: (0, 0)),
            pl.BlockSpec((D, DFF), lambda e, t: (0, 0)),
            pl.BlockSpec((DFF, D), lambda e, t: (0, 0)),
        ],
        out_specs=pl.BlockSpec(
            (TB, D), lambda e, t: (jnp.where(e == 2 * E, t, 0), 0)),
        out_shape=jax.ShapeDtypeStruct((T, D), jnp.float32),
        scratch_shapes=[
            pltpu.VMEM((T, D), jnp.bfloat16),
            pltpu.VMEM((T, 2 * E), jnp.float32),
            pltpu.VMEM((T, D), jnp.float32),
            pltpu.VMEM((D, DFF), jnp.bfloat16),
            pltpu.VMEM((D, DFF), jnp.bfloat16),
            pltpu.VMEM((DFF, D), jnp.bfloat16),
        ],
        compiler_params=pltpu.CompilerParams(
            dimension_semantics=("arbitrary", "arbitrary")),
    )(x, maskf, gates, e_score_correction_bias,
      text_wg, text_wu, text_wd, vision_wg, vision_wu, vision_wd,
      shared_wg, shared_wu, shared_wd)

    return out.reshape(orig_shape)


# speedup vs baseline: 2.4925x; 1.4442x over previous
import jax
import jax.numpy as jnp
from jax import lax
from jax.experimental import pallas as pl
from jax.experimental.pallas import tpu as pltpu

T = 2048
D = 1024
DFF = 512
E = 8


def _gating(xb, gates, bias_t, bias_v, mask_col):
    logits = lax.dot_general(
        xb, gates.astype(jnp.bfloat16), (((1,), (0,)), ((), ())),
        preferred_element_type=jnp.float32,
    )

    def bank(l, brow):
        m = jnp.max(l, axis=1, keepdims=True)
        ex = jnp.exp(l - m)
        s = ex / jnp.sum(ex, axis=1, keepdims=True)
        b = s + brow
        iota = lax.broadcasted_iota(jnp.int32, (T, E), 1)
        m1 = jnp.max(b, axis=1, keepdims=True)
        idx1 = jnp.min(jnp.where(b == m1, iota, E), axis=1, keepdims=True)
        bm = jnp.where(iota == idx1, -jnp.inf, b)
        m2 = jnp.max(bm, axis=1, keepdims=True)
        idx2 = jnp.min(jnp.where(bm == m2, iota, E), axis=1, keepdims=True)
        w1 = jnp.sum(jnp.where(iota == idx1, s, 0.0), axis=1, keepdims=True)
        w2 = jnp.sum(jnp.where(iota == idx2, s, 0.0), axis=1, keepdims=True)
        ws = w1 + w2
        return (iota == idx1) * (w1 / ws) + (iota == idx2) * (w2 / ws)

    wt = bank(logits[:, :E], bias_t) * (1.0 - mask_col)
    wv = bank(logits[:, E:], bias_v) * mask_col
    return jnp.concatenate([wt, wv], axis=1)


def _moe_body(xb_ref, maskf_ref, gates_ref, bias_ref,
              twg_ref, twu_ref, twd_ref,
              vwg_ref, vwu_ref, vwd_ref,
              swg_ref, swu_ref, swd_ref,
              out_ref, w_scr):
    e = pl.program_id(0)
    xb = xb_ref[...]

    @pl.when(e == 0)
    def _():
        w_scr[...] = _gating(xb, gates_ref[...], bias_ref[0:1, :],
                             bias_ref[1:2, :], maskf_ref[...])
        out_ref[...] = jnp.zeros((T, D), jnp.float32)

    iota16 = lax.broadcasted_iota(jnp.int32, (T, 2 * E), 1)
    wcol = jnp.sum(w_scr[...] * (iota16 == e).astype(jnp.float32),
                   axis=1, keepdims=True)
    wcol = jnp.where(e == 2 * E, 1.0, wcol)

    CH = 512

    def mlp_acc(wg, wu, wd):
        wgb = wg.astype(jnp.bfloat16)
        wub = wu.astype(jnp.bfloat16)
        wdb = wd.astype(jnp.bfloat16)
        for c in range(T // CH):
            sl = pl.ds(c * CH, CH)
            xc = xb_ref[sl, :]
            g = lax.dot_general(xc, wgb, (((1,), (0,)), ((), ())),
                                preferred_element_type=jnp.float32)
            u = lax.dot_general(xc, wub, (((1,), (0,)), ((), ())),
                                preferred_element_type=jnp.float32)
            a = (g * jax.nn.sigmoid(g)) * u
            h = lax.dot_general(a.astype(jnp.bfloat16), wdb,
                                (((1,), (0,)), ((), ())),
                                preferred_element_type=jnp.float32)
            out_ref[sl, :] += wcol[c * CH:(c + 1) * CH, :] * h

    @pl.when(e < E)
    def _():
        mlp_acc(twg_ref[0], twu_ref[0], twd_ref[0])

    @pl.when((e >= E) & (e < 2 * E))
    def _():
        mlp_acc(vwg_ref[0], vwu_ref[0], vwd_ref[0])

    @pl.when(e == 2 * E)
    def _():
        mlp_acc(swg_ref[...], swu_ref[...], swd_ref[...])


def kernel(hidden_states, visual_token_mask, text_gate_w, vision_gate_w,
           e_score_correction_bias, text_wg, text_wu, text_wd,
           vision_wg, vision_wu, vision_wd, shared_wg, shared_wu, shared_wd):
    orig_shape = hidden_states.shape
    xb = hidden_states.reshape(T, D).astype(jnp.bfloat16)
    maskf = visual_token_mask.reshape(T, 1).astype(jnp.float32)
    gates = jnp.concatenate([text_gate_w, vision_gate_w], axis=1)

    out = pl.pallas_call(
        _moe_body,
        grid=(2 * E + 1,),
        in_specs=[
            pl.BlockSpec((T, D), lambda e: (0, 0)),
            pl.BlockSpec((T, 1), lambda e: (0, 0)),
            pl.BlockSpec((D, 2 * E), lambda e: (0, 0)),
            pl.BlockSpec((2, E), lambda e: (0, 0)),
            pl.BlockSpec((1, D, DFF), lambda e: (jnp.minimum(e, E - 1), 0, 0)),
            pl.BlockSpec((1, D, DFF), lambda e: (jnp.minimum(e, E - 1), 0, 0)),
            pl.BlockSpec((1, DFF, D), lambda e: (jnp.minimum(e, E - 1), 0, 0)),
            pl.BlockSpec((1, D, DFF),
                         lambda e: (jnp.clip(e - E, 0, E - 1), 0, 0)),
            pl.BlockSpec((1, D, DFF),
                         lambda e: (jnp.clip(e - E, 0, E - 1), 0, 0)),
            pl.BlockSpec((1, DFF, D),
                         lambda e: (jnp.clip(e - E, 0, E - 1), 0, 0)),
            pl.BlockSpec((D, DFF), lambda e: (0, 0)),
            pl.BlockSpec((D, DFF), lambda e: (0, 0)),
            pl.BlockSpec((DFF, D), lambda e: (0, 0)),
        ],
        out_specs=pl.BlockSpec((T, D), lambda e: (0, 0)),
        out_shape=jax.ShapeDtypeStruct((T, D), jnp.float32),
        scratch_shapes=[
            pltpu.VMEM((T, 2 * E), jnp.float32),
        ],
        compiler_params=pltpu.CompilerParams(
            dimension_semantics=("arbitrary",)),
    )(xb, maskf, gates, e_score_correction_bias,
      text_wg, text_wu, text_wd, vision_wg, vision_wu, vision_wd,
      shared_wg, shared_wu, shared_wd)

    return out.reshape(orig_shape)


# speedup vs baseline: 2.6704x; 1.0714x over previous
import functools

import jax
import jax.numpy as jnp
from jax import lax
from jax.experimental import pallas as pl
from jax.experimental.pallas import tpu as pltpu
from jax.experimental.pallas import tpu_sc as plsc

T = 2048
D = 1024
DFF = 512
E = 8
B = 256
NSLOT = 2 * T + 16 * B
NBLK = NSLOT // B
NW = 32
TPW = T // NW


def _bank_topk(logits, brow):
    m = jnp.max(logits, axis=1, keepdims=True)
    ex = jnp.exp(logits - m)
    s = ex / jnp.sum(ex, axis=1, keepdims=True)
    b = s + brow
    iota = lax.broadcasted_iota(jnp.int32, (T, E), 1)
    m1 = jnp.max(b, axis=1, keepdims=True)
    idx1 = jnp.min(jnp.where(b == m1, iota, E), axis=1, keepdims=True)
    bm = jnp.where(iota == idx1, -jnp.inf, b)
    m2 = jnp.max(bm, axis=1, keepdims=True)
    idx2 = jnp.min(jnp.where(bm == m2, iota, E), axis=1, keepdims=True)
    w1 = jnp.sum(jnp.where(iota == idx1, s, 0.0), axis=1, keepdims=True)
    w2 = jnp.sum(jnp.where(iota == idx2, s, 0.0), axis=1, keepdims=True)
    ws = w1 + w2
    return idx1, idx2, w1 / ws, w2 / ws


def _plan_body(xb_ref, maskf_ref, gates_ref, bias_ref,
               p0_ref, p1_ref, w0_ref, w1_ref, be_ref):
    xb = xb_ref[...]
    logits = lax.dot_general(
        xb, gates_ref[...].astype(jnp.bfloat16), (((1,), (0,)), ((), ())),
        preferred_element_type=jnp.float32)
    ti1, ti2, tw1, tw2 = _bank_topk(logits[:, :E], bias_ref[0:1, :])
    vi1, vi2, vw1, vw2 = _bank_topk(logits[:, E:], bias_ref[1:2, :])
    mask = maskf_ref[...] > 0.5
    g1 = jnp.where(mask, vi1 + E, ti1)
    g2 = jnp.where(mask, vi2 + E, ti2)
    w0_ref[...] = jnp.where(mask, vw1, tw1)
    w1_ref[...] = jnp.where(mask, vw2, tw2)

    CH = 512
    iota16 = lax.broadcasted_iota(jnp.int32, (CH, 16), 1)
    r = lax.broadcasted_iota(jnp.int32, (CH, CH), 0)
    c = lax.broadcasted_iota(jnp.int32, (CH, CH), 1)
    lstrict = (c < r).astype(jnp.bfloat16)
    carry = jnp.zeros((1, 16), jnp.float32)
    pfx_chunks = []
    oh_chunks = []
    for ci in range(T // CH):
        sl = slice(ci * CH, (ci + 1) * CH)
        oh = ((iota16 == g1[sl]) .astype(jnp.float32)
              + (iota16 == g2[sl]).astype(jnp.float32))
        pfx = lax.dot_general(lstrict, oh.astype(jnp.bfloat16),
                              (((1,), (0,)), ((), ())),
                              preferred_element_type=jnp.float32)
        pfx_chunks.append(pfx + carry)
        oh_chunks.append(oh)
        carry = carry + jnp.sum(oh, axis=0, keepdims=True)

    cnt = carry
    pc = jnp.ceil(cnt / B) * B
    ir = lax.broadcasted_iota(jnp.int32, (16, 16), 0)
    ic = lax.broadcasted_iota(jnp.int32, (16, 16), 1)
    mtri = (ir < ic).astype(jnp.bfloat16)
    off = lax.dot_general(pc.astype(jnp.bfloat16), mtri,
                          (((1,), (0,)), ((), ())),
                          preferred_element_type=jnp.float32)

    for ci in range(T // CH):
        sl = slice(ci * CH, (ci + 1) * CH)
        base = off + pfx_chunks[ci]
        i16 = lax.broadcasted_iota(jnp.int32, (CH, 16), 1)
        p0 = jnp.sum(jnp.where(i16 == g1[sl], base, 0.0), axis=1,
                     keepdims=True)
        p1 = jnp.sum(jnp.where(i16 == g2[sl], base + (g1[sl] == g2[sl]),
                               0.0), axis=1, keepdims=True)
        p0_ref[ci * CH:(ci + 1) * CH, :] = p0.astype(jnp.int32)
        p1_ref[ci * CH:(ci + 1) * CH, :] = p1.astype(jnp.int32)

    bi = lax.broadcasted_iota(jnp.int32, (1, 128), 1)
    s = (bi * B).astype(jnp.float32)
    acc = jnp.zeros((1, 128), jnp.float32)
    for e in range(16):
        acc = acc + (lax.slice(off, (0, e), (1, e + 1)) <= s).astype(
            jnp.float32)
    used = (lax.slice(off, (0, 15), (1, 16))
            + lax.slice(pc, (0, 15), (1, 16))) / B
    be = jnp.where(s / B < used, acc - 1.0, 16.0)
    be_ref[...] = be.astype(jnp.int32)


def _plan_call(xb, maskf, gates, bias):
    return pl.pallas_call(
        _plan_body,
        out_shape=[
            jax.ShapeDtypeStruct((T, 1), jnp.int32),
            jax.ShapeDtypeStruct((T, 1), jnp.int32),
            jax.ShapeDtypeStruct((T, 1), jnp.float32),
            jax.ShapeDtypeStruct((T, 1), jnp.float32),
            jax.ShapeDtypeStruct((1, 128), jnp.int32),
        ],
    )(xb, maskf, gates, bias)


def _sc_scatter_call(xf, p0, p1):
    mesh = plsc.VectorSubcoreMesh(core_axis_name="c", subcore_axis_name="s")

    @functools.partial(
        pl.kernel, mesh=mesh,
        out_type=jax.ShapeDtypeStruct((NSLOT, D), jnp.float32),
        scratch_types=[
            pltpu.VMEM((TPW,), jnp.int32),
            pltpu.VMEM((TPW,), jnp.int32),
            pltpu.VMEM((TPW, D), jnp.float32),
            pltpu.SemaphoreType.DMA,
        ],
    )
    def k(xb_hbm, p0_hbm, p1_hbm, xs_hbm, i0_v, i1_v, rows_v, sem):
        wid = lax.axis_index("s") * 2 + lax.axis_index("c")
        base = wid * TPW
        pltpu.sync_copy(p0_hbm.at[pl.ds(base, TPW)], i0_v)
        pltpu.sync_copy(p1_hbm.at[pl.ds(base, TPW)], i1_v)
        pltpu.sync_copy(xb_hbm.at[pl.ds(base, TPW)], rows_v)
        pltpu.async_copy(rows_v, xs_hbm.at[i0_v], sem).wait()
        pltpu.async_copy(rows_v, xs_hbm.at[i1_v], sem).wait()

    return k(xf, p0, p1)


def _sc_gather_call(y, p0, p1):
    mesh = plsc.VectorSubcoreMesh(core_axis_name="c", subcore_axis_name="s")

    @functools.partial(
        pl.kernel, mesh=mesh,
        out_type=[jax.ShapeDtypeStruct((T, D), jnp.float32),
                  jax.ShapeDtypeStruct((T, D), jnp.float32)],
        scratch_types=[
            pltpu.VMEM((TPW,), jnp.int32),
            pltpu.VMEM((TPW, D), jnp.float32),
            pltpu.SemaphoreType.DMA,
        ],
    )
    def k(y_hbm, p0_hbm, p1_hbm, y0_hbm, y1_hbm, idx_v, rows_v, sem):
        wid = lax.axis_index("s") * 2 + lax.axis_index("c")
        base = wid * TPW
        pltpu.sync_copy(p0_hbm.at[pl.ds(base, TPW)], idx_v)
        pltpu.async_copy(y_hbm.at[idx_v], rows_v, sem).wait()
        pltpu.sync_copy(rows_v, y0_hbm.at[pl.ds(base, TPW)])
        pltpu.sync_copy(p1_hbm.at[pl.ds(base, TPW)], idx_v)
        pltpu.async_copy(y_hbm.at[idx_v], rows_v, sem).wait()
        pltpu.sync_copy(rows_v, y1_hbm.at[pl.ds(base, TPW)])

    return k(y, p0, p1)


def _mlp_f32w(xc, wg, wu, wd):
    g = lax.dot_general(xc, wg.astype(jnp.bfloat16), (((1,), (0,)), ((), ())),
                        preferred_element_type=jnp.float32)
    u = lax.dot_general(xc, wu.astype(jnp.bfloat16), (((1,), (0,)), ((), ())),
                        preferred_element_type=jnp.float32)
    a = (g * jax.nn.sigmoid(g)) * u
    return lax.dot_general(a.astype(jnp.bfloat16), wd.astype(jnp.bfloat16),
                           (((1,), (0,)), ((), ())),
                           preferred_element_type=jnp.float32)


def _gmm_body(be_sref, xs_ref,
              twg_ref, twu_ref, twd_ref, vwg_ref, vwu_ref, vwd_ref,
              y_ref):
    i = pl.program_id(0)
    be = be_sref[0, i]

    xc = xs_ref[...].astype(jnp.bfloat16)

    @pl.when(be < E)
    def _():
        y_ref[...] = _mlp_f32w(xc, twg_ref[0], twu_ref[0], twd_ref[0])

    @pl.when((be >= E) & (be < 2 * E))
    def _():
        y_ref[...] = _mlp_f32w(xc, vwg_ref[0], vwu_ref[0], vwd_ref[0])


def _gmm_call(be, xs, twg, twu, twd, vwg, vwu, vwd):
    grid_spec = pltpu.PrefetchScalarGridSpec(
        num_scalar_prefetch=1,
        grid=(NBLK,),
        in_specs=[
            pl.BlockSpec((B, D), lambda i, be_s: (i, 0)),
            pl.BlockSpec((1, D, DFF),
                         lambda i, be_s: (jnp.clip(be_s[0, i], 0, E - 1), 0, 0)),
            pl.BlockSpec((1, D, DFF),
                         lambda i, be_s: (jnp.clip(be_s[0, i], 0, E - 1), 0, 0)),
            pl.BlockSpec((1, DFF, D),
                         lambda i, be_s: (jnp.clip(be_s[0, i], 0, E - 1), 0, 0)),
            pl.BlockSpec((1, D, DFF),
                         lambda i, be_s: (jnp.clip(be_s[0, i] - E, 0, E - 1), 0, 0)),
            pl.BlockSpec((1, D, DFF),
                         lambda i, be_s: (jnp.clip(be_s[0, i] - E, 0, E - 1), 0, 0)),
            pl.BlockSpec((1, DFF, D),
                         lambda i, be_s: (jnp.clip(be_s[0, i] - E, 0, E - 1), 0, 0)),
        ],
        out_specs=pl.BlockSpec((B, D), lambda i, be_s: (i, 0)),
    )
    return pl.pallas_call(
        _gmm_body,
        grid_spec=grid_spec,
        out_shape=jax.ShapeDtypeStruct((NSLOT, D), jnp.float32),
        compiler_params=pltpu.CompilerParams(
            dimension_semantics=("arbitrary",)),
    )(be, xs, twg, twu, twd, vwg, vwu, vwd)


def _shared_body(xb_ref, swg_ref, swu_ref, swd_ref, out_ref):
    out_ref[...] = _mlp_f32w(xb_ref[...], swg_ref[...], swu_ref[...],
                             swd_ref[...])


def _shared_call(xb, swg, swu, swd):
    return pl.pallas_call(
        _shared_body,
        grid=(4,),
        in_specs=[
            pl.BlockSpec((T // 4, D), lambda i: (i, 0)),
            pl.BlockSpec((D, DFF), lambda i: (0, 0)),
            pl.BlockSpec((D, DFF), lambda i: (0, 0)),
            pl.BlockSpec((DFF, D), lambda i: (0, 0)),
        ],
        out_specs=pl.BlockSpec((T // 4, D), lambda i: (i, 0)),
        out_shape=jax.ShapeDtypeStruct((T, D), jnp.float32),
        compiler_params=pltpu.CompilerParams(
            dimension_semantics=("arbitrary",)),
    )(xb, swg, swu, swd)


def _combine_body(y0_ref, y1_ref, w0_ref, w1_ref, sh_ref, out_ref):
    out_ref[...] = (w0_ref[...] * y0_ref[...] + w1_ref[...] * y1_ref[...]
                    + sh_ref[...])


def _combine_call(y0, y1, w0, w1, sh):
    C = T // 4
    return pl.pallas_call(
        _combine_body,
        grid=(4,),
        in_specs=[
            pl.BlockSpec((C, D), lambda i: (i, 0)),
            pl.BlockSpec((C, D), lambda i: (i, 0)),
            pl.BlockSpec((C, 1), lambda i: (i, 0)),
            pl.BlockSpec((C, 1), lambda i: (i, 0)),
            pl.BlockSpec((C, D), lambda i: (i, 0)),
        ],
        out_specs=pl.BlockSpec((C, D), lambda i: (i, 0)),
        out_shape=jax.ShapeDtypeStruct((T, D), jnp.float32),
        compiler_params=pltpu.CompilerParams(
            dimension_semantics=("arbitrary",)),
    )(y0, y1, w0, w1, sh)


def kernel(hidden_states, visual_token_mask, text_gate_w, vision_gate_w,
           e_score_correction_bias, text_wg, text_wu, text_wd,
           vision_wg, vision_wu, vision_wd, shared_wg, shared_wu, shared_wd):
    orig_shape = hidden_states.shape
    xb = hidden_states.reshape(T, D).astype(jnp.bfloat16)
    maskf = visual_token_mask.reshape(T, 1).astype(jnp.float32)
    gates = jnp.concatenate([text_gate_w, vision_gate_w], axis=1)

    p0, p1, w0, w1, be = _plan_call(xb, maskf, gates, e_score_correction_bias)
    p0f = p0.reshape(T)
    p1f = p1.reshape(T)

    xf = hidden_states.reshape(T, D)
    xs = _sc_scatter_call(xf, p0f, p1f)
    sh = _shared_call(xb, shared_wg, shared_wu, shared_wd)
    y = _gmm_call(be.reshape(1, 128), xs,
                  text_wg, text_wu, text_wd, vision_wg, vision_wu, vision_wd)
    y0, y1 = _sc_gather_call(y, p0f, p1f)
    out = _combine_call(y0, y1, w0, w1, sh)
    return out.reshape(orig_shape)
